# baseline (speedup 1.0000x reference)
import functools

import jax
import jax.numpy as jnp
from jax import lax
from jax.experimental import pallas as pl
from jax.experimental.pallas import tpu as pltpu
from jax.experimental.pallas import tpu_sc as plsc

_NC = 2
_NS = 16
_NW = _NC * _NS
_LANES = 128

_VT = 2048
_RC = 8


def _pass_a_body(nv, v, w_ref, wt_ref, bo_ref, wg_ref, bg_ref, c1_ref, m_ref, s_ref):
    j = pl.program_id(0)

    @pl.when(j == 0)
    def _():
        m_ref[...] = jnp.full_like(m_ref, -1e30)
        s_ref[...] = jnp.zeros_like(s_ref)

    wb = w_ref[...].astype(jnp.bfloat16)
    Wb = wt_ref[...].astype(jnp.bfloat16)
    logits = lax.dot_general(wb, Wb, (((1,), (0,)), ((), ())),
                             preferred_element_type=jnp.float32)
    logits = logits + bo_ref[...]
    col = j * _VT + lax.broadcasted_iota(jnp.int32, logits.shape, 1)
    logits = jnp.where(col < v, logits, -1e30)

    m_old = m_ref[...]
    m_new = jnp.maximum(m_old, jnp.max(logits, axis=1, keepdims=True))
    s_ref[...] = (s_ref[...] * jnp.exp(m_old - m_new)
                  + jnp.sum(jnp.exp(logits - m_new), axis=1, keepdims=True))
    m_ref[...] = m_new

    @pl.when(j == nv - 1)
    def _():
        z = lax.dot_general(w_ref[...], wg_ref[...], (((1,), (0,)), ((), ())),
                            preferred_element_type=jnp.float32) + bg_ref[...]
        c1_ref[...] = m_ref[...] + jnp.log(s_ref[...]) - jax.nn.log_sigmoid(z)


def _pass_b_body(w_ref, wt_ref, bo_ref, c1_ref, out_ref):
    wb = w_ref[...].astype(jnp.bfloat16)
    Wb = wt_ref[...].astype(jnp.bfloat16)
    logits = lax.dot_general(wb, Wb, (((1,), (0,)), ((), ())),
                             preferred_element_type=jnp.float32)
    out_ref[...] = logits + bo_ref[...] - c1_ref[...]


def _dedup_body(v, ids_ref, att_ref, w_ref, wg_ref, bg_ref, pos_ref, contrib_ref):
    i = pl.program_id(0)
    ids = ids_ref[...]
    att = att_ref[...]
    eq = (ids[:, :, None] == ids[:, None, :]).astype(jnp.float32)
    summed = jnp.sum(eq * att[:, None, :], axis=2)
    z = lax.dot_general(w_ref[...], wg_ref[...], (((1,), (0,)), ((), ())),
                        preferred_element_type=jnp.float32) + bg_ref[...]
    contrib_ref[...] = summed * jax.nn.sigmoid(-z)
    row = i * _RC + lax.broadcasted_iota(jnp.int32, ids.shape, 0)
    pos_ref[...] = ids + row * v


def _merge_body(old_ref, contrib_ref, new_ref):
    new_ref[...] = jnp.log(jnp.exp(old_ref[...]) + contrib_ref[...])


def _sc_gather_body(nchunks, dense_hbm, pos_hbm, out_hbm, idx_v, val_v, sem):
    wid = lax.axis_index("s") * _NC + lax.axis_index("c")
    pltpu.sync_copy(pos_hbm.at[wid], idx_v)
    copies = [
        pltpu.async_copy(dense_hbm.at[idx_v.at[j]], val_v.at[j], sem)
        for j in range(nchunks)
    ]
    for c in copies:
        c.wait()
    pltpu.sync_copy(val_v, out_hbm.at[wid])


def _sc_scatter_body(nchunks, dense_ref, pos_hbm, val_hbm, idx_v, val_v, sem):
    wid = lax.axis_index("s") * _NC + lax.axis_index("c")
    pltpu.sync_copy(pos_hbm.at[wid], idx_v)
    pltpu.sync_copy(val_hbm.at[wid], val_v)
    copies = [
        pltpu.async_copy(val_v.at[j], dense_ref.at[idx_v.at[j]], sem)
        for j in range(nchunks)
    ]
    for c in copies:
        c.wait()


def kernel(wrapper_outputs, attention_scores, memory_src_ids, W_out, b_out, W_gen, b_gen):
    B, L = wrapper_outputs.shape
    S = attention_scores.shape[1]
    V = W_out.shape[1]
    nv = pl.cdiv(V, _VT)
    ids = memory_src_ids.astype(jnp.int32)
    b_out2 = b_out.reshape(1, V)
    b_gen2 = b_gen.reshape(1, 1)

    c1 = pl.pallas_call(
        functools.partial(_pass_a_body, nv, V),
        grid=(nv,),
        in_specs=[
            pl.BlockSpec((B, L), lambda j: (0, 0)),
            pl.BlockSpec((L, _VT), lambda j: (0, j)),
            pl.BlockSpec((1, _VT), lambda j: (0, j)),
            pl.BlockSpec((L, 1), lambda j: (0, 0)),
            pl.BlockSpec((1, 1), lambda j: (0, 0)),
        ],
        out_specs=pl.BlockSpec((B, 1), lambda j: (0, 0)),
        out_shape=jax.ShapeDtypeStruct((B, 1), jnp.float32),
        scratch_shapes=[
            pltpu.VMEM((B, 1), jnp.float32),
            pltpu.VMEM((B, 1), jnp.float32),
        ],
    )(wrapper_outputs, W_out, b_out2, W_gen, b_gen2)

    dense = pl.pallas_call(
        _pass_b_body,
        grid=(nv,),
        in_specs=[
            pl.BlockSpec((B, L), lambda j: (0, 0)),
            pl.BlockSpec((L, _VT), lambda j: (0, j)),
            pl.BlockSpec((1, _VT), lambda j: (0, j)),
            pl.BlockSpec((B, 1), lambda j: (0, 0)),
        ],
        out_specs=pl.BlockSpec((B, _VT), lambda j: (0, j)),
        out_shape=jax.ShapeDtypeStruct((B, V), jnp.float32),
    )(wrapper_outputs, W_out, b_out2, c1)

    pos, contrib = pl.pallas_call(
        functools.partial(_dedup_body, V),
        grid=(B // _RC,),
        in_specs=[
            pl.BlockSpec((_RC, S), lambda i: (i, 0)),
            pl.BlockSpec((_RC, S), lambda i: (i, 0)),
            pl.BlockSpec((_RC, L), lambda i: (i, 0)),
            pl.BlockSpec((L, 1), lambda i: (0, 0)),
            pl.BlockSpec((1, 1), lambda i: (0, 0)),
        ],
        out_specs=[
            pl.BlockSpec((_RC, S), lambda i: (i, 0)),
            pl.BlockSpec((_RC, S), lambda i: (i, 0)),
        ],
        out_shape=[
            jax.ShapeDtypeStruct((B, S), jnp.int32),
            jax.ShapeDtypeStruct((B, S), jnp.float32),
        ],
    )(ids, attention_scores, wrapper_outputs, W_gen, b_gen2)

    total = B * S
    assert total % (_NW * _LANES) == 0
    nchunks = total // (_NW * _LANES)
    pos3 = pos.reshape(_NW, nchunks, _LANES)
    contrib3 = contrib.reshape(_NW, nchunks, _LANES)
    dense_flat = dense.reshape(B * V)

    mesh = plsc.VectorSubcoreMesh(core_axis_name="c", subcore_axis_name="s",
                                  num_cores=_NC, num_subcores=_NS)

    gather = pl.kernel(
        functools.partial(_sc_gather_body, nchunks),
        out_type=jax.ShapeDtypeStruct((_NW, nchunks, _LANES), jnp.float32),
        mesh=mesh,
        scratch_types=[
            pltpu.VMEM((nchunks, _LANES), jnp.int32),
            pltpu.VMEM((nchunks, _LANES), jnp.float32),
            pltpu.SemaphoreType.DMA,
        ],
    )
    old3 = gather(dense_flat, pos3)

    new3 = pl.pallas_call(
        _merge_body,
        grid=(1,),
        in_specs=[
            pl.BlockSpec((_NW * nchunks, _LANES), lambda i: (0, 0)),
            pl.BlockSpec((_NW * nchunks, _LANES), lambda i: (0, 0)),
        ],
        out_specs=pl.BlockSpec((_NW * nchunks, _LANES), lambda i: (0, 0)),
        out_shape=jax.ShapeDtypeStruct((_NW * nchunks, _LANES), jnp.float32),
    )(old3.reshape(_NW * nchunks, _LANES), contrib3.reshape(_NW * nchunks, _LANES))

    scatter = pl.kernel(
        functools.partial(_sc_scatter_body, nchunks),
        out_type=(),
        mesh=mesh,
        scratch_types=[
            pltpu.VMEM((nchunks, _LANES), jnp.int32),
            pltpu.VMEM((nchunks, _LANES), jnp.float32),
            pltpu.SemaphoreType.DMA,
        ],
    )
    dense_ref = jax.new_ref(dense_flat)
    scatter(dense_ref, pos3, new3.reshape(_NW, nchunks, _LANES))
    return dense_ref[...].reshape(B, V)
